# baseline (speedup 1.0000x reference)
import functools

import jax
import jax.numpy as jnp
from jax import lax
from jax.experimental import pallas as pl
from jax.experimental.pallas import tpu as pltpu
from jax.experimental.pallas import tpu_sc as plsc

VOCAB_ = 1000000
DIM_ = 32
BATCH_ = 4096
HIST_ = 200

_NC = 2
_NS = 16
_NW = _NC * _NS
_RPW = BATCH_ // _NW
_SPLIT = 128
_REM = HIST_ - _SPLIT


def _tanh16(x):
    e = jnp.exp(-2.0 * jnp.abs(x))
    t = (1.0 - e) / (1.0 + e)
    return jnp.where(x < 0.0, -t, t)


_R = 4
_U = 8
_NCHUNK = _RPW // _R


def _sc_body(table_hbm, idx_hbm, w_hbm, out_hbm,
             idx_v, w_v, buf0, buf1, out_v, sem0, sem1):
    wid = lax.axis_index("s") * _NC + lax.axis_index("c")
    base = wid * _RPW
    pltpu.sync_copy(idx_hbm.at[pl.ds(base * HIST_, _RPW * HIST_)], idx_v)
    pltpu.sync_copy(w_hbm, w_v)

    bufs = (buf0, buf1)
    sems = (sem0, sem1)

    def descs(chunk, buf, sem):
        ds = []
        for r in range(_R):
            o = (chunk * _R + r) * HIST_
            ds.append(pltpu.make_async_copy(
                table_hbm.at[idx_v.at[pl.ds(o, _SPLIT)]],
                buf.at[r, pl.ds(0, _SPLIT)], sem))
            ds.append(pltpu.make_async_copy(
                table_hbm.at[idx_v.at[pl.ds(o + _SPLIT, _REM)]],
                buf.at[r, pl.ds(_SPLIT, _REM)], sem))
        return ds

    def fire(chunk, buf, sem):
        for d in descs(chunk, buf, sem):
            d.start()

    def drain(chunk, buf, sem):
        for d in descs(chunk, buf, sem):
            d.wait()

    def compute(chunk, buf):
        z = jnp.zeros((16,), jnp.float32)

        def t_body(i, accs):
            a = list(accs)
            for u in range(_U):
                t = i * _U + u
                w = w_v[t, :]
                for r in range(_R):
                    a[2 * r] = a[2 * r] + w * buf[r, t, pl.ds(0, 16)]
                    a[2 * r + 1] = a[2 * r + 1] + w * buf[r, t, pl.ds(16, 16)]
            return tuple(a)

        accs = lax.fori_loop(0, HIST_ // _U, t_body, tuple([z] * (2 * _R)))
        for r in range(_R):
            row = chunk * _R + r
            out_v[row, pl.ds(0, 16)] = _tanh16(accs[2 * r])
            out_v[row, pl.ds(16, 16)] = _tanh16(accs[2 * r + 1])

    fire(0, buf0, sem0)

    @pl.loop(0, _NCHUNK, step=2)
    def _(c):
        for b in range(2):
            chunk = c + b
            drain(chunk, bufs[b], sems[b])

            @pl.when(chunk + 1 < _NCHUNK)
            def _():
                fire(chunk + 1, bufs[1 - b], sems[1 - b])

            compute(chunk, bufs[b])

    pltpu.sync_copy(out_v, out_hbm.at[pl.ds(base, _RPW)])


_NTC = 7812
_VUP = 7813 * 128


def _sc_transpose_body(tableT_hbm, tail_hbm, ttab_hbm,
                       tin0, tin1, tout0, tout1, tailv,
                       sin0, sin1, sout0, sout1):
    wid = lax.axis_index("s") * _NC + lax.axis_index("c")
    tins = (tin0, tin1)
    touts = (tout0, tout1)
    sins = (sin0, sin1)
    souts = (sout0, sout1)

    rows0 = lax.iota(jnp.int32, 16)
    rows1 = rows0 + 16
    NB = _NTC // (4 * _NW) + 1
    NBT = _NTC // 4

    def in_desc(m, b):
        return pltpu.make_async_copy(
            tableT_hbm.at[pl.ds(0, DIM_), pl.ds(m * 512, 512)],
            tins[b].at[pl.ds(0, DIM_), pl.ds(0, 512)], sins[b])

    def out_desc(m, b):
        return pltpu.make_async_copy(
            touts[b], ttab_hbm.at[pl.ds(m * 16384, 16384)], souts[b])

    in_desc(wid, 0).start()

    @pl.loop(0, 2 * (NB // 2 + 2), step=2)
    def _(nn):
        for b in range(2):
            n = nn + b
            m = wid + n * _NW

            @pl.when(jnp.logical_and(m >= 2 * _NW, m - 2 * _NW < NBT))
            def _():
                out_desc(m - 2 * _NW, b).wait()

            @pl.when(m < NBT)
            def _():
                in_desc(m, b).wait()

                @pl.when(m + _NW < NBT)
                def _():
                    in_desc(m + _NW, 1 - b).start()

                def col_chunk(q, _):
                    for cc in range(128):
                        c = q * 128 + cc
                        cols = jnp.full((16,), 1, jnp.int32) * c
                        v0 = plsc.load_gather(tins[b], [rows0, cols])
                        v1 = plsc.load_gather(tins[b], [rows1, cols])
                        touts[b][pl.ds(c * DIM_, 16)] = v0
                        touts[b][pl.ds(c * DIM_ + 16, 16)] = v1
                    return 0

                lax.fori_loop(0, 4, col_chunk, 0)
                out_desc(m, b).start()

    @pl.when(wid == _NW - 1)
    def _():
        pltpu.sync_copy(tail_hbm, tailv)
        pltpu.sync_copy(tailv, ttab_hbm.at[pl.ds(_NTC * 4096, 2048)])


def _sc_transpose(tableT, tail_lin):
    fn = pl.kernel(
        _sc_transpose_body,
        out_type=jax.ShapeDtypeStruct((_VUP * DIM_,), jnp.float32),
        mesh=plsc.VectorSubcoreMesh(core_axis_name="c", subcore_axis_name="s"),
        compiler_params=pltpu.CompilerParams(use_tc_tiling_on_sc=True,
                                             needs_layout_passes=False),
        scratch_types=[
            pltpu.VMEM((DIM_, 513), jnp.float32),
            pltpu.VMEM((DIM_, 513), jnp.float32),
            pltpu.VMEM((16384,), jnp.float32),
            pltpu.VMEM((16384,), jnp.float32),
            pltpu.VMEM((2048,), jnp.float32),
            pltpu.SemaphoreType.DMA,
            pltpu.SemaphoreType.DMA,
            pltpu.SemaphoreType.DMA,
            pltpu.SemaphoreType.DMA,
        ],
    )
    return fn(tableT, tail_lin)


def kernel(indices, table, ssm_decay):
    d = jax.nn.sigmoid(ssm_decay.astype(jnp.float32))
    pows = jnp.cumprod(jnp.full((HIST_,), d, dtype=jnp.float32))
    pdesc = jnp.concatenate([jnp.ones((1,), jnp.float32), pows[:-1]])
    w = (1.0 - d) * pdesc[::-1]
    wexp = jnp.broadcast_to(w[:, None], (HIST_, 16)).astype(jnp.float32)

    inds_flat = indices.reshape(-1).astype(jnp.int32)

    tail_lin = table[_NTC * 128:, :].reshape(-1)
    table_rm = _sc_transpose(table.T, tail_lin).reshape(_VUP, DIM_)

    fn = pl.kernel(
        _sc_body,
        out_type=jax.ShapeDtypeStruct((BATCH_, DIM_), jnp.float32),
        mesh=plsc.VectorSubcoreMesh(core_axis_name="c", subcore_axis_name="s"),
        compiler_params=pltpu.CompilerParams(use_tc_tiling_on_sc=False),
        scratch_types=[
            pltpu.VMEM((_RPW * HIST_,), jnp.int32),
            pltpu.VMEM((HIST_, 16), jnp.float32),
            pltpu.VMEM((_R, HIST_, DIM_), jnp.float32),
            pltpu.VMEM((_R, HIST_, DIM_), jnp.float32),
            pltpu.VMEM((_RPW, DIM_), jnp.float32),
            pltpu.SemaphoreType.DMA,
            pltpu.SemaphoreType.DMA,
        ],
    )
    return fn(table_rm, inds_flat, wexp)


# speedup vs baseline: 2.2968x; 2.2968x over previous
import functools

import jax
import jax.numpy as jnp
from jax import lax
from jax.experimental import pallas as pl
from jax.experimental.pallas import tpu as pltpu
from jax.experimental.pallas import tpu_sc as plsc

VOCAB_ = 1000000
DIM_ = 32
BATCH_ = 4096
HIST_ = 200

_NC = 2
_NS = 16
_NW = _NC * _NS
_RPW = BATCH_ // _NW
_SPLIT = 128
_REM = HIST_ - _SPLIT


def _tanh16(x):
    e = jnp.exp(-2.0 * jnp.abs(x))
    t = (1.0 - e) / (1.0 + e)
    return jnp.where(x < 0.0, -t, t)


_R = 4
_U = 8
_NCHUNK = _RPW // _R


def _sc_body(table_hbm, idx_hbm, w_hbm, out_hbm,
             idx_v, w_v, buf0, buf1, out_v, sem0, sem1):
    wid = lax.axis_index("s") * _NC + lax.axis_index("c")
    base = wid * _RPW
    pltpu.sync_copy(idx_hbm.at[pl.ds(base * HIST_, _RPW * HIST_)], idx_v)
    pltpu.sync_copy(w_hbm, w_v)

    bufs = (buf0, buf1)
    sems = (sem0, sem1)

    def descs(chunk, buf, sem):
        ds = []
        for r in range(_R):
            o = (chunk * _R + r) * HIST_
            ds.append(pltpu.make_async_copy(
                table_hbm.at[idx_v.at[pl.ds(o, _SPLIT)]],
                buf.at[r, pl.ds(0, _SPLIT)], sem))
            ds.append(pltpu.make_async_copy(
                table_hbm.at[idx_v.at[pl.ds(o + _SPLIT, _REM)]],
                buf.at[r, pl.ds(_SPLIT, _REM)], sem))
        return ds

    def fire(chunk, buf, sem):
        for d in descs(chunk, buf, sem):
            d.start()

    def drain(chunk, buf, sem):
        for d in descs(chunk, buf, sem):
            d.wait()

    def compute(chunk, buf):
        z = jnp.zeros((16,), jnp.float32)

        def t_body(i, accs):
            a = list(accs)
            for u in range(_U):
                t = i * _U + u
                w = w_v[t, :]
                for r in range(_R):
                    a[2 * r] = a[2 * r] + w * buf[r, t, pl.ds(0, 16)]
                    a[2 * r + 1] = a[2 * r + 1] + w * buf[r, t, pl.ds(16, 16)]
            return tuple(a)

        accs = lax.fori_loop(0, HIST_ // _U, t_body, tuple([z] * (2 * _R)))
        for r in range(_R):
            row = chunk * _R + r
            out_v[row, pl.ds(0, 16)] = _tanh16(accs[2 * r])
            out_v[row, pl.ds(16, 16)] = _tanh16(accs[2 * r + 1])

    fire(0, buf0, sem0)

    @pl.loop(0, _NCHUNK, step=2)
    def _(c):
        for b in range(2):
            chunk = c + b
            drain(chunk, bufs[b], sems[b])

            @pl.when(chunk + 1 < _NCHUNK)
            def _():
                fire(chunk + 1, bufs[1 - b], sems[1 - b])

            compute(chunk, bufs[b])

    pltpu.sync_copy(out_v, out_hbm.at[pl.ds(base, _RPW)])


_TBK = 8192
_TGRID = -(-VOCAB_ // _TBK)
_VPAD = _TGRID * _TBK


def _transpose_body(xt_ref, y_ref):
    x = xt_ref[...]
    eye = jax.lax.broadcasted_iota(jnp.int32, (DIM_, DIM_), 0)
    eye = jnp.where(eye == jax.lax.broadcasted_iota(jnp.int32, (DIM_, DIM_), 1),
                    jnp.float32(1), jnp.float32(0))
    parts = [
        jax.lax.dot_general(
            x[:, 512 * q:512 * (q + 1)], eye, (((0,), (0,)), ((), ())),
            preferred_element_type=jnp.float32)
        for q in range(_TBK // 512)
    ]
    rows = [
        jnp.concatenate(parts[4 * p:4 * p + 4], axis=1)
        for p in range(_TBK // 2048)
    ]
    y_ref[...] = jnp.concatenate(rows, axis=0)


def _transpose_table(tableT):
    return pl.pallas_call(
        _transpose_body,
        grid=(_TGRID,),
        in_specs=[pl.BlockSpec((DIM_, _TBK), lambda g: (0, g))],
        out_specs=pl.BlockSpec((_TBK // 4, 128), lambda g: (g, 0)),
        out_shape=jax.ShapeDtypeStruct((_VPAD // 4, 128), jnp.float32),
    )(tableT)


def kernel(indices, table, ssm_decay):
    d = jax.nn.sigmoid(ssm_decay.astype(jnp.float32))
    pows = jnp.cumprod(jnp.full((HIST_,), d, dtype=jnp.float32))
    pdesc = jnp.concatenate([jnp.ones((1,), jnp.float32), pows[:-1]])
    w = (1.0 - d) * pdesc[::-1]
    wexp = jnp.broadcast_to(w[:, None], (HIST_, 16)).astype(jnp.float32)

    i = indices.reshape(-1).astype(jnp.int32)
    inds_flat = (i & ~jnp.int32(2047)) | ((i & 511) << 2) | ((i >> 9) & 3)

    table_rm = _transpose_table(table.T).reshape(_VPAD, DIM_)

    fn = pl.kernel(
        _sc_body,
        out_type=jax.ShapeDtypeStruct((BATCH_, DIM_), jnp.float32),
        mesh=plsc.VectorSubcoreMesh(core_axis_name="c", subcore_axis_name="s"),
        compiler_params=pltpu.CompilerParams(use_tc_tiling_on_sc=False),
        scratch_types=[
            pltpu.VMEM((_RPW * HIST_,), jnp.int32),
            pltpu.VMEM((HIST_, 16), jnp.float32),
            pltpu.VMEM((_R, HIST_, DIM_), jnp.float32),
            pltpu.VMEM((_R, HIST_, DIM_), jnp.float32),
            pltpu.VMEM((_RPW, DIM_), jnp.float32),
            pltpu.SemaphoreType.DMA,
            pltpu.SemaphoreType.DMA,
        ],
    )
    return fn(table_rm, inds_flat, wexp)


# speedup vs baseline: 2.3244x; 1.0120x over previous
import functools

import jax
import jax.numpy as jnp
from jax import lax
from jax.experimental import pallas as pl
from jax.experimental.pallas import tpu as pltpu
from jax.experimental.pallas import tpu_sc as plsc

VOCAB_ = 1000000
DIM_ = 32
BATCH_ = 4096
HIST_ = 200

_NC = 2
_NS = 16
_NW = _NC * _NS
_RPW = BATCH_ // _NW
_SPLIT = 128
_REM = HIST_ - _SPLIT


def _tanh16(x):
    e = jnp.exp(-2.0 * jnp.abs(x))
    t = (1.0 - e) / (1.0 + e)
    return jnp.where(x < 0.0, -t, t)


_R = 4
_U = 8
_NCHUNK = _RPW // _R


def _sc_body(table_hbm, idx_hbm, w_hbm, out_hbm,
             idx_v, w_v, buf0, buf1, out_v, sem0, sem1):
    wid = lax.axis_index("s") * _NC + lax.axis_index("c")
    base = wid * _RPW
    pltpu.sync_copy(idx_hbm.at[pl.ds(base * HIST_, _RPW * HIST_)], idx_v)
    pltpu.sync_copy(w_hbm, w_v)

    bufs = (buf0, buf1)
    sems = (sem0, sem1)

    def descs(chunk, buf, sem):
        ds = []
        for r in range(_R):
            o = (chunk * _R + r) * HIST_
            ds.append(pltpu.make_async_copy(
                table_hbm.at[idx_v.at[pl.ds(o, _SPLIT)]],
                buf.at[r, pl.ds(0, _SPLIT)], sem))
            ds.append(pltpu.make_async_copy(
                table_hbm.at[idx_v.at[pl.ds(o + _SPLIT, _REM)]],
                buf.at[r, pl.ds(_SPLIT, _REM)], sem))
        return ds

    def fire(chunk, buf, sem):
        for d in descs(chunk, buf, sem):
            d.start()

    def drain(chunk, buf, sem):
        for d in descs(chunk, buf, sem):
            d.wait()

    def compute(chunk, buf):
        z = jnp.zeros((16,), jnp.float32)

        def t_body(i, accs):
            a = list(accs)
            for u in range(_U):
                t = i * _U + u
                w = w_v[t, :]
                for r in range(_R):
                    a[2 * r] = a[2 * r] + w * buf[r, t, pl.ds(0, 16)]
                    a[2 * r + 1] = a[2 * r + 1] + w * buf[r, t, pl.ds(16, 16)]
            return tuple(a)

        accs = lax.fori_loop(0, HIST_ // _U, t_body, tuple([z] * (2 * _R)))
        for r in range(_R):
            row = chunk * _R + r
            out_v[row, pl.ds(0, 16)] = _tanh16(accs[2 * r])
            out_v[row, pl.ds(16, 16)] = _tanh16(accs[2 * r + 1])

    fire(0, buf0, sem0)

    @pl.loop(0, _NCHUNK, step=2)
    def _(c):
        for b in range(2):
            chunk = c + b
            drain(chunk, bufs[b], sems[b])

            @pl.when(chunk + 1 < _NCHUNK)
            def _():
                fire(chunk + 1, bufs[1 - b], sems[1 - b])

            compute(chunk, bufs[b])

    pltpu.sync_copy(out_v, out_hbm.at[pl.ds(base, _RPW)])


_TBK = 8192
_TGRID = -(-VOCAB_ // _TBK)
_VPAD = _TGRID * _TBK


def _transpose_body(xt_ref, y_ref):
    x = xt_ref[...]
    parts = [
        jnp.swapaxes(x[:, 512 * q:512 * (q + 1)], 0, 1)
        for q in range(_TBK // 512)
    ]
    rows = [
        jnp.concatenate(parts[4 * p:4 * p + 4], axis=1)
        for p in range(_TBK // 2048)
    ]
    y_ref[...] = jnp.concatenate(rows, axis=0)


def _transpose_table(tableT):
    return pl.pallas_call(
        _transpose_body,
        grid=(_TGRID,),
        in_specs=[pl.BlockSpec((DIM_, _TBK), lambda g: (0, g))],
        out_specs=pl.BlockSpec((_TBK // 4, 128), lambda g: (g, 0)),
        out_shape=jax.ShapeDtypeStruct((_VPAD // 4, 128), jnp.float32),
    )(tableT)


def kernel(indices, table, ssm_decay):
    d = jax.nn.sigmoid(ssm_decay.astype(jnp.float32))
    pows = jnp.cumprod(jnp.full((HIST_,), d, dtype=jnp.float32))
    pdesc = jnp.concatenate([jnp.ones((1,), jnp.float32), pows[:-1]])
    w = (1.0 - d) * pdesc[::-1]
    wexp = jnp.broadcast_to(w[:, None], (HIST_, 16)).astype(jnp.float32)

    i = indices.reshape(-1).astype(jnp.int32)
    inds_flat = (i & ~jnp.int32(2047)) | ((i & 511) << 2) | ((i >> 9) & 3)

    table_rm = _transpose_table(table.T).reshape(_VPAD, DIM_)

    fn = pl.kernel(
        _sc_body,
        out_type=jax.ShapeDtypeStruct((BATCH_, DIM_), jnp.float32),
        mesh=plsc.VectorSubcoreMesh(core_axis_name="c", subcore_axis_name="s"),
        compiler_params=pltpu.CompilerParams(use_tc_tiling_on_sc=False),
        scratch_types=[
            pltpu.VMEM((_RPW * HIST_,), jnp.int32),
            pltpu.VMEM((HIST_, 16), jnp.float32),
            pltpu.VMEM((_R, HIST_, DIM_), jnp.float32),
            pltpu.VMEM((_R, HIST_, DIM_), jnp.float32),
            pltpu.VMEM((_RPW, DIM_), jnp.float32),
            pltpu.SemaphoreType.DMA,
            pltpu.SemaphoreType.DMA,
        ],
    )
    return fn(table_rm, inds_flat, wexp)


# speedup vs baseline: 2.3505x; 1.0112x over previous
import functools

import jax
import jax.numpy as jnp
from jax import lax
from jax.experimental import pallas as pl
from jax.experimental.pallas import tpu as pltpu
from jax.experimental.pallas import tpu_sc as plsc

VOCAB_ = 1000000
DIM_ = 32
BATCH_ = 4096
HIST_ = 200

_NC = 2
_NS = 16
_NW = _NC * _NS
_RPW = BATCH_ // _NW
_SPLIT = 128
_REM = HIST_ - _SPLIT


def _tanh16(x):
    e = jnp.exp(-2.0 * jnp.abs(x))
    t = (1.0 - e) / (1.0 + e)
    return jnp.where(x < 0.0, -t, t)


_R = 4
_U = 8
_NCHUNK = _RPW // _R


def _sc_body(table_hbm, idx_hbm, w_hbm, out_hbm,
             idx_v, w_v, buf0, buf1, out_v, sem0, sem1):
    wid = lax.axis_index("s") * _NC + lax.axis_index("c")
    base = wid * _RPW
    pltpu.sync_copy(idx_hbm.at[pl.ds(base * HIST_, _RPW * HIST_)], idx_v)
    pltpu.sync_copy(w_hbm, w_v)

    bufs = (buf0, buf1)
    sems = (sem0, sem1)

    def descs(chunk, buf, sem):
        ds = []
        for r in range(_R):
            o = (chunk * _R + r) * HIST_
            ds.append(pltpu.make_async_copy(
                table_hbm.at[idx_v.at[pl.ds(o, _SPLIT)]],
                buf.at[r, pl.ds(0, _SPLIT)], sem))
            ds.append(pltpu.make_async_copy(
                table_hbm.at[idx_v.at[pl.ds(o + _SPLIT, _REM)]],
                buf.at[r, pl.ds(_SPLIT, _REM)], sem))
        return ds

    def fire(chunk, buf, sem):
        for d in descs(chunk, buf, sem):
            d.start()

    def drain(chunk, buf, sem):
        for d in descs(chunk, buf, sem):
            d.wait()

    def compute(chunk, buf):
        z = jnp.zeros((16,), jnp.float32)

        def t_body(i, accs):
            a = list(accs)
            for u in range(_U):
                t = i * _U + u
                w = w_v[t, :]
                for r in range(_R):
                    a[2 * r] = a[2 * r] + w * buf[r, t, pl.ds(0, 16)]
                    a[2 * r + 1] = a[2 * r + 1] + w * buf[r, t, pl.ds(16, 16)]
            return tuple(a)

        accs = lax.fori_loop(0, HIST_ // _U, t_body, tuple([z] * (2 * _R)))
        for r in range(_R):
            row = chunk * _R + r
            out_v[row, pl.ds(0, 16)] = _tanh16(accs[2 * r])
            out_v[row, pl.ds(16, 16)] = _tanh16(accs[2 * r + 1])

    fire(0, buf0, sem0)

    @pl.loop(0, _NCHUNK, step=2)
    def _(c):
        for b in range(2):
            chunk = c + b
            drain(chunk, bufs[b], sems[b])

            @pl.when(chunk + 1 < _NCHUNK)
            def _():
                fire(chunk + 1, bufs[1 - b], sems[1 - b])

            compute(chunk, bufs[b])

    pltpu.sync_copy(out_v, out_hbm.at[pl.ds(base, _RPW)])


_TBK = 16384
_TGRID = -(-VOCAB_ // _TBK)
_VPAD = _TGRID * _TBK


def _transpose_body(xt_ref, y_ref):
    x = xt_ref[...]
    parts = [
        jnp.swapaxes(x[:, 512 * q:512 * (q + 1)], 0, 1)
        for q in range(_TBK // 512)
    ]
    rows = [
        jnp.concatenate(parts[4 * p:4 * p + 4], axis=1)
        for p in range(_TBK // 2048)
    ]
    y_ref[...] = jnp.concatenate(rows, axis=0)


def _transpose_table(tableT):
    return pl.pallas_call(
        _transpose_body,
        grid=(_TGRID,),
        in_specs=[pl.BlockSpec((DIM_, _TBK), lambda g: (0, g))],
        out_specs=pl.BlockSpec((_TBK // 4, 128), lambda g: (g, 0)),
        out_shape=jax.ShapeDtypeStruct((_VPAD // 4, 128), jnp.float32),
    )(tableT)


def kernel(indices, table, ssm_decay):
    d = jax.nn.sigmoid(ssm_decay.astype(jnp.float32))
    pows = jnp.cumprod(jnp.full((HIST_,), d, dtype=jnp.float32))
    pdesc = jnp.concatenate([jnp.ones((1,), jnp.float32), pows[:-1]])
    w = (1.0 - d) * pdesc[::-1]
    wexp = jnp.broadcast_to(w[:, None], (HIST_, 16)).astype(jnp.float32)

    i = indices.reshape(-1).astype(jnp.int32)
    inds_flat = (i & ~jnp.int32(2047)) | ((i & 511) << 2) | ((i >> 9) & 3)

    table_rm = _transpose_table(table.T).reshape(_VPAD, DIM_)

    fn = pl.kernel(
        _sc_body,
        out_type=jax.ShapeDtypeStruct((BATCH_, DIM_), jnp.float32),
        mesh=plsc.VectorSubcoreMesh(core_axis_name="c", subcore_axis_name="s"),
        compiler_params=pltpu.CompilerParams(use_tc_tiling_on_sc=False),
        scratch_types=[
            pltpu.VMEM((_RPW * HIST_,), jnp.int32),
            pltpu.VMEM((HIST_, 16), jnp.float32),
            pltpu.VMEM((_R, HIST_, DIM_), jnp.float32),
            pltpu.VMEM((_R, HIST_, DIM_), jnp.float32),
            pltpu.VMEM((_RPW, DIM_), jnp.float32),
            pltpu.SemaphoreType.DMA,
            pltpu.SemaphoreType.DMA,
        ],
    )
    return fn(table_rm, inds_flat, wexp)
